# baseline (speedup 1.0000x reference)
import dataclasses as _dc

import jax
import jax.numpy as jnp
from jax import lax
from jax.experimental import pallas as pl
from jax.experimental.pallas import tpu as pltpu
from jax.experimental.pallas import tpu_sc as plsc

N = 10000
NP = 10240
H = 64
HP = 128
E = 320000
G = 16
NC, NS = 2, 16
NW = NC * NS
C = 80
CH = 128
ET = C * CH
EP = NW * ET
CS = 16

_mesh = plsc.VectorSubcoreMesh(core_axis_name="c", subcore_axis_name="s")

_sc_params = pltpu.CompilerParams()
if "needs_layout_passes" in pltpu.CompilerParams.__dataclass_fields__:
    _sc_params = _dc.replace(_sc_params, needs_layout_passes=False)


def _deg_body(col_hbm, ea_hbm, out_hbm, colv, eav, part, tmp, accv, shared):
    cid = lax.axis_index("c")
    sid = lax.axis_index("s")
    wid = sid * NC + cid
    pltpu.sync_copy(col_hbm.at[wid], colv)
    pltpu.sync_copy(ea_hbm.at[wid], eav)
    z16 = jnp.zeros((16,), jnp.float32)

    @pl.loop(0, C)
    def _zero(j):
        for k in range(8):
            part[j, pl.ds(k * 16, 16)] = z16

    @pl.loop(0, C)
    def _accum(j):
        for g in range(8):
            sl = pl.ds(g * 16, 16)
            c16 = colv[j, sl]
            w16 = jnp.abs(eav[j, sl])
            hi = lax.shift_right_logical(c16, 7)
            lo = lax.bitwise_and(c16, 127)
            plsc.addupdate_scatter(part, [hi, lo], w16)

    pltpu.sync_copy(part, shared.at[sid])
    plsc.subcore_barrier()

    @pl.when(sid < 10)
    def _reduce():
        rs = sid * 8
        for j in range(8):
            for k in range(8):
                accv[j, pl.ds(k * 16, 16)] = z16
        for t in range(NS):
            pltpu.sync_copy(shared.at[t, pl.ds(rs, 8)], tmp)
            for j in range(8):
                for k in range(8):
                    sl = pl.ds(k * 16, 16)
                    accv[j, sl] = accv[j, sl] + tmp[j, sl]
        pltpu.sync_copy(accv, out_hbm.at[cid, pl.ds(rs, 8)])


def _deg(colp, eap):
    return pl.kernel(
        _deg_body,
        out_type=jax.ShapeDtypeStruct((NC, C, CH), jnp.float32),
        mesh=_mesh,
        compiler_params=_sc_params,
        scratch_types=[
            pltpu.VMEM((C, CH), jnp.int32),
            pltpu.VMEM((C, CH), jnp.float32),
            pltpu.VMEM((C, CH), jnp.float32),
            pltpu.VMEM((8, CH), jnp.float32),
            pltpu.VMEM((8, CH), jnp.float32),
            pltpu.VMEM_SHARED((NS, C, CH), jnp.float32),
        ],
    )(colp, eap)


def _norm_body(row_hbm, col_hbm, ea_hbm, dinv_hbm, out_hbm,
               rowv, colv, eav, dinvv):
    cid = lax.axis_index("c")
    sid = lax.axis_index("s")
    wid = sid * NC + cid
    pltpu.sync_copy(row_hbm.at[wid], rowv)
    pltpu.sync_copy(col_hbm.at[wid], colv)
    pltpu.sync_copy(ea_hbm.at[wid], eav)
    pltpu.sync_copy(dinv_hbm, dinvv)

    @pl.loop(0, C)
    def _norm(j):
        for g in range(8):
            sl = pl.ds(g * 16, 16)
            r16 = rowv[j, sl]
            c16 = colv[j, sl]
            w16 = jnp.abs(eav[j, sl])
            n16 = plsc.load_gather(dinvv, [r16]) * w16 \
                * plsc.load_gather(dinvv, [c16])
            eav[j, sl] = n16

    pltpu.sync_copy(eav, out_hbm.at[wid])


def _edge_norm(rowp, colp, eap, dinv_flat):
    return pl.kernel(
        _norm_body,
        out_type=jax.ShapeDtypeStruct((NW, C, CH), jnp.float32),
        mesh=_mesh,
        compiler_params=_sc_params,
        scratch_types=[
            pltpu.VMEM((C, CH), jnp.int32),
            pltpu.VMEM((C, CH), jnp.int32),
            pltpu.VMEM((C, CH), jnp.float32),
            pltpu.VMEM((NP,), jnp.float32),
        ],
    )(rowp, colp, eap, dinv_flat)


def _mp_body(y_hbm, row_hbm, col_hbm, norm_hbm, out_hbm,
             rowv, colv, normv, gb0, gb1, acc, gsem, ssem):
    gbufs = (gb0, gb1)
    cid = lax.axis_index("c")
    sid = lax.axis_index("s")
    wid = sid * NC + cid
    z16 = jnp.zeros((16,), jnp.float32)

    @pl.loop(0, CH)
    def _zero(e):
        for k in range(HP // 16):
            gb0[e, pl.ds(k * 16, 16)] = z16

    for k in range(5):
        pltpu.sync_copy(gb0, acc.at[pl.ds(sid * 640 + k * 128, 128)])
    plsc.subcore_barrier()

    def scale(b, l):
        lf = jnp.full((16,), l, jnp.int32)

        @pl.loop(0, CH, unroll=8)
        def _scale(e):
            ef = jnp.full((16,), e, jnp.int32)
            n16 = plsc.load_gather(normv, [lf, ef])
            for k in range(H // 16):
                sl = pl.ds(k * 16, 16)
                gbufs[b][e, sl] = gbufs[b][e, sl] * n16

    def wait_gather(b):
        pltpu.make_async_copy(y_hbm.at[rowv.at[b]], gbufs[b],
                              gsem.at[b]).wait()

    def wait_scatter(b):
        pltpu.make_async_copy(gbufs[b], acc.at[colv.at[b]],
                              ssem.at[b]).wait()

    for h in range(C // CS):
        hs = h * CS
        pltpu.sync_copy(row_hbm.at[wid, pl.ds(hs, CS)], rowv)
        pltpu.sync_copy(col_hbm.at[wid, pl.ds(hs, CS)], colv)
        pltpu.sync_copy(norm_hbm.at[wid, pl.ds(hs, CS)], normv)

        for b in range(2):
            pltpu.async_copy(y_hbm.at[rowv.at[b]], gbufs[b], gsem.at[b])

        @pl.loop(0, CS - 2, step=2)
        def _ring(lo):
            for b in range(2):
                l = lo + b
                wait_gather(b)
                scale(b, l)
                pltpu.async_copy(gbufs[b], acc.at[colv.at[l]], ssem.at[b],
                                 add=True)
                wait_scatter(b)
                pltpu.async_copy(y_hbm.at[rowv.at[l + 2]], gbufs[b],
                                 gsem.at[b])

        for b in range(2):
            l = CS - 2 + b
            wait_gather(b)
            scale(b, l)
            pltpu.async_copy(gbufs[b], acc.at[colv.at[l]], ssem.at[b],
                             add=True)
            wait_scatter(b)

    plsc.subcore_barrier()
    rs = sid * 640
    pltpu.sync_copy(acc.at[pl.ds(rs, 640)], out_hbm.at[cid, pl.ds(rs, 640)])


def _mp(y, rowp, colp, normp):
    return pl.kernel(
        _mp_body,
        out_type=jax.ShapeDtypeStruct((NC, NP, HP), jnp.float32),
        mesh=_mesh,
        compiler_params=_sc_params,
        scratch_types=[
            pltpu.VMEM((CS, CH), jnp.int32),
            pltpu.VMEM((CS, CH), jnp.int32),
            pltpu.VMEM((CS, CH), jnp.float32),
            pltpu.VMEM((CH, HP), jnp.float32),
            pltpu.VMEM((CH, HP), jnp.float32),
            pltpu.VMEM_SHARED((NP, HP), jnp.float32),
            pltpu.SemaphoreType.DMA((2,)),
            pltpu.SemaphoreType.DMA((2,)),
        ],
    )(y, rowp, colp, normp)


def _rsqrt_body(a_ref, b_ref, o_ref):
    o_ref[...] = lax.rsqrt(a_ref[...] + b_ref[...] + 1.0)


def _rsqrt_tc(a, b):
    return pl.pallas_call(
        _rsqrt_body,
        out_shape=jax.ShapeDtypeStruct((C, CH), jnp.float32),
    )(a, b)


def _mm_body(x_ref, w_ref, o_ref):
    w = w_ref[...]
    wp = jnp.concatenate([w, jnp.zeros((w.shape[0], HP - H), w.dtype)], axis=1)
    o_ref[...] = jnp.dot(x_ref[...], wp, preferred_element_type=jnp.float32)


def _mm(x, W):
    return pl.pallas_call(
        _mm_body,
        out_shape=jax.ShapeDtypeStruct((x.shape[0], HP), jnp.float32),
    )(x, W)


def _layer_body(sa_ref, sb_ref, xw_ref, dinv_ref, b_ref, w_ref, o_ref):
    d = dinv_ref[...]
    s = (sa_ref[...] + sb_ref[...])[:N, :H]
    h = jax.nn.relu(s + d * d * xw_ref[...][:, :H] + b_ref[...])
    w = w_ref[...]
    wp = jnp.concatenate([w, jnp.zeros((w.shape[0], HP - H), w.dtype)], axis=1)
    o_ref[...] = jnp.dot(h, wp, preferred_element_type=jnp.float32)


def _layer_tc(sa, sb, xw, dinv_col, b, W):
    return pl.pallas_call(
        _layer_body,
        out_shape=jax.ShapeDtypeStruct((N, HP), jnp.float32),
    )(sa, sb, xw, dinv_col, b, W)


def _readout_body(sa_ref, sb_ref, xw_ref, dinv_ref, b_ref, batch_ref,
                  Wa_ref, ba_ref, Wc_ref, bc_ref, logits_ref, value_ref):
    d = dinv_ref[...]
    s = (sa_ref[...] + sb_ref[...])[:N, :H]
    h = jax.nn.relu(s + d * d * xw_ref[...][:, :H] + b_ref[...])
    logits_ref[...] = jnp.dot(h, Wa_ref[...],
                              preferred_element_type=jnp.float32) + ba_ref[0, 0]
    seg = lax.broadcasted_iota(jnp.int32, (G, N), 0)
    onehot = (seg == batch_ref[...]).astype(jnp.float32)
    sums = jnp.dot(onehot, h, preferred_element_type=jnp.float32)
    counts = jnp.sum(onehot, axis=1, keepdims=True)
    mean = sums / jnp.maximum(counts, 1.0)
    value_ref[...] = jnp.dot(mean, Wc_ref[...],
                             preferred_element_type=jnp.float32) + bc_ref[0, 0]


def _readout(sa, sb, xw, dinv_col, b, batch, Wa, ba, Wc, bc):
    return pl.pallas_call(
        _readout_body,
        out_shape=(jax.ShapeDtypeStruct((N, 1), jnp.float32),
                   jax.ShapeDtypeStruct((G, 1), jnp.float32)),
    )(sa, sb, xw, dinv_col, b, batch.reshape(1, N), Wa, ba.reshape(1, 1),
      Wc, bc.reshape(1, 1))


def kernel(x, edge_index, edge_attr, batch, W1, b1, W2, b2, Wa, ba, Wc, bc):
    row = edge_index[0]
    col = edge_index[1]
    pad = EP - E
    rowp = jnp.pad(row, (0, pad)).reshape(NW, C, CH)
    colp = jnp.pad(col, (0, pad)).reshape(NW, C, CH)
    eap = jnp.pad(edge_attr, (0, pad)).reshape(NW, C, CH)

    deg_parts = _deg(colp, eap)
    dinv80 = _rsqrt_tc(deg_parts[0], deg_parts[1])
    dinv_flat = dinv80.reshape(NP)
    dinv_col = dinv_flat[:N].reshape(N, 1)
    normp = _edge_norm(rowp, colp, eap, dinv_flat)

    xw1 = _mm(x, W1)
    s1 = _mp(xw1, rowp, colp, normp)
    xw2 = _layer_tc(s1[0], s1[1], xw1, dinv_col, b1.reshape(1, H), W2)
    s2 = _mp(xw2, rowp, colp, normp)
    logits, value = _readout(s2[0], s2[1], xw2, dinv_col, b2.reshape(1, H),
                             batch, Wa, ba, Wc, bc)
    return logits.reshape(N), value


# speedup vs baseline: 1.1018x; 1.1018x over previous
import dataclasses as _dc

import jax
import jax.numpy as jnp
from jax import lax
from jax.experimental import pallas as pl
from jax.experimental.pallas import tpu as pltpu
from jax.experimental.pallas import tpu_sc as plsc

N = 10000
NP = 10240
H = 64
HP = 128
E = 320000
G = 16
NC, NS = 2, 16
NW = NC * NS
C = 80
CH = 128
ET = C * CH
EP = NW * ET
CS = 16

_mesh = plsc.VectorSubcoreMesh(core_axis_name="c", subcore_axis_name="s")

_sc_params = pltpu.CompilerParams()
if "needs_layout_passes" in pltpu.CompilerParams.__dataclass_fields__:
    _sc_params = _dc.replace(_sc_params, needs_layout_passes=False)


def _deg_body(col_hbm, ea_hbm, out_hbm, colv, eav, part, tmp, accv, shared):
    cid = lax.axis_index("c")
    sid = lax.axis_index("s")
    wid = sid * NC + cid
    pltpu.sync_copy(col_hbm.at[wid], colv)
    pltpu.sync_copy(ea_hbm.at[wid], eav)
    z16 = jnp.zeros((16,), jnp.float32)

    @pl.loop(0, C)
    def _zero(j):
        for k in range(8):
            part[j, pl.ds(k * 16, 16)] = z16

    @pl.loop(0, C)
    def _accum(j):
        for g in range(8):
            sl = pl.ds(g * 16, 16)
            c16 = colv[j, sl]
            w16 = jnp.abs(eav[j, sl])
            hi = lax.shift_right_logical(c16, 7)
            lo = lax.bitwise_and(c16, 127)
            plsc.addupdate_scatter(part, [hi, lo], w16)

    pltpu.sync_copy(part, shared.at[sid])
    plsc.subcore_barrier()

    @pl.when(sid < 10)
    def _reduce():
        rs = sid * 8
        for j in range(8):
            for k in range(8):
                accv[j, pl.ds(k * 16, 16)] = z16
        for t in range(NS):
            pltpu.sync_copy(shared.at[t, pl.ds(rs, 8)], tmp)
            for j in range(8):
                for k in range(8):
                    sl = pl.ds(k * 16, 16)
                    accv[j, sl] = accv[j, sl] + tmp[j, sl]
        pltpu.sync_copy(accv, out_hbm.at[cid, pl.ds(rs, 8)])


def _deg(colp, eap):
    return pl.kernel(
        _deg_body,
        out_type=jax.ShapeDtypeStruct((NC, C, CH), jnp.float32),
        mesh=_mesh,
        compiler_params=_sc_params,
        scratch_types=[
            pltpu.VMEM((C, CH), jnp.int32),
            pltpu.VMEM((C, CH), jnp.float32),
            pltpu.VMEM((C, CH), jnp.float32),
            pltpu.VMEM((8, CH), jnp.float32),
            pltpu.VMEM((8, CH), jnp.float32),
            pltpu.VMEM_SHARED((NS, C, CH), jnp.float32),
        ],
    )(colp, eap)


def _norm_body(row_hbm, col_hbm, ea_hbm, dinv_hbm, out_hbm,
               rowv, colv, eav, dinvv):
    cid = lax.axis_index("c")
    sid = lax.axis_index("s")
    wid = sid * NC + cid
    pltpu.sync_copy(row_hbm.at[wid], rowv)
    pltpu.sync_copy(col_hbm.at[wid], colv)
    pltpu.sync_copy(ea_hbm.at[wid], eav)
    pltpu.sync_copy(dinv_hbm, dinvv)

    @pl.loop(0, C)
    def _norm(j):
        for g in range(8):
            sl = pl.ds(g * 16, 16)
            r16 = rowv[j, sl]
            c16 = colv[j, sl]
            w16 = jnp.abs(eav[j, sl])
            n16 = plsc.load_gather(dinvv, [r16]) * w16 \
                * plsc.load_gather(dinvv, [c16])
            eav[j, sl] = n16

    pltpu.sync_copy(eav, out_hbm.at[wid])


def _edge_norm(rowp, colp, eap, dinv_flat):
    return pl.kernel(
        _norm_body,
        out_type=jax.ShapeDtypeStruct((NW, C, CH), jnp.float32),
        mesh=_mesh,
        compiler_params=_sc_params,
        scratch_types=[
            pltpu.VMEM((C, CH), jnp.int32),
            pltpu.VMEM((C, CH), jnp.int32),
            pltpu.VMEM((C, CH), jnp.float32),
            pltpu.VMEM((NP,), jnp.float32),
        ],
    )(rowp, colp, eap, dinv_flat)


NP2 = NP // 2


def _mp_body(y_hbm, row_hbm, col_hbm, norm_hbm, out_hbm,
             rowv, colv, col2v, normv, gb0, gb1, sb0, sb1, acc, gsem, ssem):
    gbufs = (gb0, gb1)
    sbufs = (sb0, sb1)
    cid = lax.axis_index("c")
    sid = lax.axis_index("s")
    wid = sid * NC + cid
    z16 = jnp.zeros((16,), jnp.float32)
    iota = lax.broadcasted_iota(jnp.int32, (16,), 0)

    @pl.loop(0, CH)
    def _zero(e):
        for k in range(HP // 16):
            gb0[e, pl.ds(k * 16, 16)] = z16

    for k in range(2):
        pltpu.sync_copy(gb0, acc.at[pl.ds(sid * 320 + k * 128, 128)])
    pltpu.sync_copy(gb0.at[pl.ds(0, 64)], acc.at[pl.ds(sid * 320 + 256, 64)])
    plsc.subcore_barrier()

    def scale(b, l):
        lf = jnp.full((16,), l, jnp.int32)

        @pl.loop(0, CH, unroll=4)
        def _scale(e):
            ef = jnp.full((16,), e, jnp.int32)
            n16 = plsc.load_gather(normv, [lf, ef])
            c16 = plsc.load_gather(colv, [lf, ef])
            par64 = lax.bitwise_and(c16, 1) * 64
            base16 = par64 + iota
            zbase16 = (64 - par64) + iota
            for k in range(H // 16):
                v16 = gbufs[b][e, pl.ds(k * 16, 16)] * n16
                plsc.store_scatter(sbufs[b], [ef, base16 + (k * 16)], v16)
            for k in range(H // 16):
                plsc.store_scatter(sbufs[b], [ef, zbase16 + (k * 16)], z16)

    def wait_gather(b):
        pltpu.make_async_copy(y_hbm.at[rowv.at[b]], gbufs[b],
                              gsem.at[b]).wait()

    def wait_scatter(b):
        pltpu.make_async_copy(sbufs[b], acc.at[col2v.at[b]],
                              ssem.at[b]).wait()

    def issue_scatter(b, l):
        pltpu.async_copy(sbufs[b], acc.at[col2v.at[l]], ssem.at[b], add=True)

    def issue_gather(b, l):
        pltpu.async_copy(y_hbm.at[rowv.at[l]], gbufs[b], gsem.at[b])

    for h in range(C // CS):
        hs = h * CS
        pltpu.sync_copy(row_hbm.at[wid, pl.ds(hs, CS)], rowv)
        pltpu.sync_copy(col_hbm.at[wid, pl.ds(hs, CS)], colv)
        pltpu.sync_copy(norm_hbm.at[wid, pl.ds(hs, CS)], normv)

        @pl.loop(0, CS)
        def _c2(j):
            for g in range(8):
                sl = pl.ds(g * 16, 16)
                col2v[j, sl] = lax.shift_right_logical(colv[j, sl], 1)

        for b in range(2):
            issue_gather(b, b)

        for b in range(2):
            wait_gather(b)
            scale(b, b)
            issue_scatter(b, b)
            issue_gather(b, b + 2)

        @pl.loop(2, CS - 2, step=2)
        def _ring(lo):
            for b in range(2):
                l = lo + b
                wait_gather(b)
                wait_scatter(b)
                scale(b, l)
                issue_scatter(b, l)
                issue_gather(b, l + 2)

        for b in range(2):
            l = CS - 2 + b
            wait_gather(b)
            wait_scatter(b)
            scale(b, l)
            issue_scatter(b, l)

        for b in range(2):
            wait_scatter(b)

    plsc.subcore_barrier()
    rs = sid * 320
    pltpu.sync_copy(acc.at[pl.ds(rs, 320)], out_hbm.at[cid, pl.ds(rs, 320)])


def _mp(y, rowp, colp, normp):
    return pl.kernel(
        _mp_body,
        out_type=jax.ShapeDtypeStruct((NC, NP2, HP), jnp.float32),
        mesh=_mesh,
        compiler_params=_sc_params,
        scratch_types=[
            pltpu.VMEM((CS, CH), jnp.int32),
            pltpu.VMEM((CS, CH), jnp.int32),
            pltpu.VMEM((CS, CH), jnp.int32),
            pltpu.VMEM((CS, CH), jnp.float32),
            pltpu.VMEM((CH, HP), jnp.float32),
            pltpu.VMEM((CH, HP), jnp.float32),
            pltpu.VMEM((CH, HP), jnp.float32),
            pltpu.VMEM((CH, HP), jnp.float32),
            pltpu.VMEM_SHARED((NP2, HP), jnp.float32),
            pltpu.SemaphoreType.DMA((2,)),
            pltpu.SemaphoreType.DMA((2,)),
        ],
    )(y, rowp, colp, normp)


def _rsqrt_body(a_ref, b_ref, o_ref):
    o_ref[...] = lax.rsqrt(a_ref[...] + b_ref[...] + 1.0)


def _rsqrt_tc(a, b):
    return pl.pallas_call(
        _rsqrt_body,
        out_shape=jax.ShapeDtypeStruct((C, CH), jnp.float32),
    )(a, b)


def _mm_body(x_ref, w_ref, o_ref):
    w = w_ref[...]
    wp = jnp.concatenate([w, jnp.zeros((w.shape[0], HP - H), w.dtype)], axis=1)
    o_ref[...] = jnp.dot(x_ref[...], wp, preferred_element_type=jnp.float32)


def _mm(x, W):
    return pl.pallas_call(
        _mm_body,
        out_shape=jax.ShapeDtypeStruct((x.shape[0], HP), jnp.float32),
    )(x, W)


def _layer_body(sa_ref, sb_ref, xw_ref, dinv_ref, b_ref, w_ref, o_ref):
    d = dinv_ref[...]
    s = (sa_ref[...] + sb_ref[...])[:N, :H]
    h = jax.nn.relu(s + d * d * xw_ref[...][:, :H] + b_ref[...])
    w = w_ref[...]
    wp = jnp.concatenate([w, jnp.zeros((w.shape[0], HP - H), w.dtype)], axis=1)
    o_ref[...] = jnp.dot(h, wp, preferred_element_type=jnp.float32)


def _layer_tc(sa, sb, xw, dinv_col, b, W):
    return pl.pallas_call(
        _layer_body,
        out_shape=jax.ShapeDtypeStruct((N, HP), jnp.float32),
    )(sa, sb, xw, dinv_col, b, W)


def _readout_body(sa_ref, sb_ref, xw_ref, dinv_ref, b_ref, batch_ref,
                  Wa_ref, ba_ref, Wc_ref, bc_ref, logits_ref, value_ref):
    d = dinv_ref[...]
    s = (sa_ref[...] + sb_ref[...])[:N, :H]
    h = jax.nn.relu(s + d * d * xw_ref[...][:, :H] + b_ref[...])
    logits_ref[...] = jnp.dot(h, Wa_ref[...],
                              preferred_element_type=jnp.float32) + ba_ref[0, 0]
    seg = lax.broadcasted_iota(jnp.int32, (G, N), 0)
    onehot = (seg == batch_ref[...]).astype(jnp.float32)
    sums = jnp.dot(onehot, h, preferred_element_type=jnp.float32)
    counts = jnp.sum(onehot, axis=1, keepdims=True)
    mean = sums / jnp.maximum(counts, 1.0)
    value_ref[...] = jnp.dot(mean, Wc_ref[...],
                             preferred_element_type=jnp.float32) + bc_ref[0, 0]


def _readout(sa, sb, xw, dinv_col, b, batch, Wa, ba, Wc, bc):
    return pl.pallas_call(
        _readout_body,
        out_shape=(jax.ShapeDtypeStruct((N, 1), jnp.float32),
                   jax.ShapeDtypeStruct((G, 1), jnp.float32)),
    )(sa, sb, xw, dinv_col, b, batch.reshape(1, N), Wa, ba.reshape(1, 1),
      Wc, bc.reshape(1, 1))


def kernel(x, edge_index, edge_attr, batch, W1, b1, W2, b2, Wa, ba, Wc, bc):
    row = edge_index[0]
    col = edge_index[1]
    pad = EP - E
    rowp = jnp.pad(row, (0, pad)).reshape(NW, C, CH)
    colp = jnp.pad(col, (0, pad)).reshape(NW, C, CH)
    eap = jnp.pad(edge_attr, (0, pad)).reshape(NW, C, CH)

    deg_parts = _deg(colp, eap)
    dinv80 = _rsqrt_tc(deg_parts[0], deg_parts[1])
    dinv_flat = dinv80.reshape(NP)
    dinv_col = dinv_flat[:N].reshape(N, 1)
    normp = _edge_norm(rowp, colp, eap, dinv_flat)

    xw1 = _mm(x, W1)
    s1 = _mp(xw1, rowp, colp, normp).reshape(NC, NP, H)
    xw2 = _layer_tc(s1[0], s1[1], xw1, dinv_col, b1.reshape(1, H), W2)
    s2 = _mp(xw2, rowp, colp, normp).reshape(NC, NP, H)
    logits, value = _readout(s2[0], s2[1], xw2, dinv_col, b2.reshape(1, H),
                             batch, Wa, ba, Wc, bc)
    return logits.reshape(N), value
